# baseline (speedup 1.0000x reference)
import functools

import jax
import jax.numpy as jnp
from jax import lax
from jax.experimental import pallas as pl
from jax.experimental.pallas import tpu as pltpu
from jax.experimental.pallas import tpu_sc as plsc

B = 10000
DEG = 32
N = 50000
D = 128
K = DEG + 1
L = 16
NC, NS = 2, 16
G = 64
NG0, NG1 = 10, 0
BPAD = NS * (NG0 + NG1) * G
B0 = NS * NG0 * G
NR0, NR1 = NG0 * K, NG1 * K
VSEG = D // L
NACC = 4
CU = 4


def _sc_aggregate(idx0, features):
    mesh = plsc.VectorSubcoreMesh(
        core_axis_name="c", subcore_axis_name="s", num_cores=NC, num_subcores=NS
    )

    @functools.partial(
        pl.kernel,
        out_type=jax.ShapeDtypeStruct((BPAD, D), jnp.float32),
        mesh=mesh,
        scratch_types=[
            pltpu.VMEM((NR0, G), jnp.int32),
            [pltpu.VMEM((G, D), jnp.float32) for _ in range(NACC)],
            [pltpu.SemaphoreType.DMA for _ in range(NACC)],
        ],
    )
    def body(idx0_hbm, feat_hbm, out_hbm, idx_v, accs, sems):
        c = lax.axis_index("c")
        s = lax.axis_index("s")

        @pl.when(c == 0)
        def _():
            pltpu.sync_copy(idx0_hbm.at[s], idx_v)

        ng = lax.select(c == 0, NG0, NG1)
        obase = lax.select(c == 0, s * (NG0 * G), B0 + s * (NG1 * G))
        scale = jnp.float32(1.0 / K)
        nfull = (K - NACC) // NACC
        ntail = (K - NACC) % NACC

        def g_body(g, carry):
            r0 = g * K
            for k in range(NACC):
                pltpu.async_copy(feat_hbm.at[idx_v.at[r0 + k]], accs[k], sems[k])

            def t_body(t, carry2):
                for k in range(NACC):
                    pltpu.make_async_copy(
                        feat_hbm.at[idx_v.at[r0]], accs[k], sems[k]
                    ).wait()
                    j = NACC + NACC * t + k
                    pltpu.async_copy(
                        feat_hbm.at[idx_v.at[r0 + j]], accs[k], sems[k], add=True
                    )
                return carry2

            lax.fori_loop(0, nfull, t_body, 0)
            for k in range(ntail):
                pltpu.make_async_copy(
                    feat_hbm.at[idx_v.at[r0]], accs[k], sems[k]
                ).wait()
                j = NACC + NACC * nfull + k
                pltpu.async_copy(
                    feat_hbm.at[idx_v.at[r0 + j]], accs[k], sems[k], add=True
                )
            for k in range(NACC):
                pltpu.make_async_copy(
                    feat_hbm.at[idx_v.at[r0]], accs[k], sems[k]
                ).wait()

            def red_body(c4, carry3):
                for cc in range(CU):
                    for v in range(VSEG):
                        sl = pl.ds(v * L, L)
                        acc = accs[0][c4 * CU + cc, sl]
                        for k in range(1, NACC):
                            acc = acc + accs[k][c4 * CU + cc, sl]
                        accs[0][c4 * CU + cc, sl] = acc * scale
                return carry3

            lax.fori_loop(0, G // CU, red_body, 0)
            pltpu.sync_copy(accs[0], out_hbm.at[pl.ds(obase + g * G, G)])
            return carry

        lax.fori_loop(0, ng, g_body, 0)

    return body(idx0, features)


def kernel(nodes, adj, features):
    nodes = nodes.astype(jnp.int32)
    adj = adj.astype(jnp.int32)
    idx = jnp.concatenate([nodes[:, None], adj], axis=1)
    idx = jnp.pad(idx, ((0, BPAD - B), (0, 0)))
    idx0 = idx[:B0].reshape(NS, NG0, G, K).transpose(0, 1, 3, 2).reshape(NS, NR0, G)
    out = _sc_aggregate(idx0, features)
    return out[:B]


# speedup vs baseline: 3.9201x; 3.9201x over previous
import functools

import jax
import jax.numpy as jnp
from jax import lax
from jax.experimental import pallas as pl
from jax.experimental.pallas import tpu as pltpu
from jax.experimental.pallas import tpu_sc as plsc

B = 10000
DEG = 32
N = 50000
D = 128
K = DEG + 1
L = 16
NC, NS = 2, 16
NW = NC * NS
G = 64
BPAD = ((B + NW * G - 1) // (NW * G)) * (NW * G)
BPW = BPAD // NW
NG = BPW // G
NR = NG * K
VSEG = D // L
NACC = 4
CU = 4


def _sc_aggregate(idx, features):
    mesh = plsc.VectorSubcoreMesh(
        core_axis_name="c", subcore_axis_name="s", num_cores=NC, num_subcores=NS
    )

    @functools.partial(
        pl.kernel,
        out_type=jax.ShapeDtypeStruct((BPAD, D), jnp.float32),
        mesh=mesh,
        scratch_types=[
            pltpu.VMEM((NR, G), jnp.int32),
            [pltpu.VMEM((G, D), jnp.float32) for _ in range(NACC)],
            [pltpu.SemaphoreType.DMA for _ in range(NACC)],
        ],
    )
    def body(idx_hbm, feat_hbm, out_hbm, idx_v, accs, sems):
        w = lax.axis_index("s") * NC + lax.axis_index("c")
        pltpu.sync_copy(idx_hbm.at[w], idx_v)
        scale = jnp.float32(1.0 / K)
        nfull = (K - NACC) // NACC
        ntail = (K - NACC) % NACC

        def g_body(g, carry):
            r0 = g * K
            for k in range(NACC):
                pltpu.async_copy(feat_hbm.at[idx_v.at[r0 + k]], accs[k], sems[k])

            def t_body(t, carry2):
                for k in range(NACC):
                    pltpu.make_async_copy(
                        feat_hbm.at[idx_v.at[r0]], accs[k], sems[k]
                    ).wait()
                    j = NACC + NACC * t + k
                    pltpu.async_copy(
                        feat_hbm.at[idx_v.at[r0 + j]], accs[k], sems[k], add=True
                    )
                return carry2

            lax.fori_loop(0, nfull, t_body, 0)
            for k in range(ntail):
                pltpu.make_async_copy(
                    feat_hbm.at[idx_v.at[r0]], accs[k], sems[k]
                ).wait()
                j = NACC + NACC * nfull + k
                pltpu.async_copy(
                    feat_hbm.at[idx_v.at[r0 + j]], accs[k], sems[k], add=True
                )
            for k in range(NACC):
                pltpu.make_async_copy(
                    feat_hbm.at[idx_v.at[r0]], accs[k], sems[k]
                ).wait()

            def red_body(c4, carry3):
                for cc in range(CU):
                    for v in range(VSEG):
                        sl = pl.ds(v * L, L)
                        acc = accs[0][c4 * CU + cc, sl]
                        for k in range(1, NACC):
                            acc = acc + accs[k][c4 * CU + cc, sl]
                        accs[0][c4 * CU + cc, sl] = acc * scale
                return carry3

            lax.fori_loop(0, G // CU, red_body, 0)
            pltpu.sync_copy(accs[0], out_hbm.at[pl.ds(w * BPW + g * G, G)])
            return carry

        lax.fori_loop(0, NG, g_body, 0)

    return body(idx, features)


def kernel(nodes, adj, features):
    nodes = nodes.astype(jnp.int32)
    adj = adj.astype(jnp.int32)
    idx = jnp.concatenate([nodes[:, None], adj], axis=1)
    pad = (
        jnp.arange(BPAD - B, dtype=jnp.int32)[:, None]
        + G * jnp.arange(K, dtype=jnp.int32)[None, :]
    )
    idx = jnp.concatenate([idx, pad], axis=0)
    idx = idx.reshape(NW, NG, G, K).transpose(0, 1, 3, 2).reshape(NW, NR, G)
    out = _sc_aggregate(idx, features)
    return out[:B]


# speedup vs baseline: 4.0533x; 1.0340x over previous
import functools

import jax
import jax.numpy as jnp
from jax import lax
from jax.experimental import pallas as pl
from jax.experimental.pallas import tpu as pltpu
from jax.experimental.pallas import tpu_sc as plsc

B = 10000
DEG = 32
N = 50000
D = 128
K = DEG + 1
L = 16
NC, NS = 2, 16
NW = NC * NS
G = 64
BPAD = ((B + NW * G - 1) // (NW * G)) * (NW * G)
BPW = BPAD // NW
NG = BPW // G
NR = NG * K
VSEG = D // L
NACC = 4
CU = 4
REM = B % G


def _sc_aggregate(idx, features):
    mesh = plsc.VectorSubcoreMesh(
        core_axis_name="c", subcore_axis_name="s", num_cores=NC, num_subcores=NS
    )

    @functools.partial(
        pl.kernel,
        out_type=jax.ShapeDtypeStruct((B, D), jnp.float32),
        mesh=mesh,
        scratch_types=[
            pltpu.VMEM((NR, G), jnp.int32),
            [pltpu.VMEM((G, D), jnp.float32) for _ in range(NACC)],
            [pltpu.SemaphoreType.DMA for _ in range(NACC)],
        ],
    )
    def body(idx_hbm, feat_hbm, out_hbm, idx_v, accs, sems):
        w = lax.axis_index("s") * NC + lax.axis_index("c")
        pltpu.sync_copy(idx_hbm.at[w], idx_v)
        scale = jnp.float32(1.0 / K)
        ng = lax.min(NG, lax.max(0, (B - w * BPW + G - 1) // G))
        nfull = (K - NACC) // NACC
        ntail = (K - NACC) % NACC

        def g_body(g, carry):
            r0 = g * K
            for k in range(NACC):
                pltpu.async_copy(feat_hbm.at[idx_v.at[r0 + k]], accs[k], sems[k])

            def t_body(t, carry2):
                for k in range(NACC):
                    pltpu.make_async_copy(
                        feat_hbm.at[idx_v.at[r0]], accs[k], sems[k]
                    ).wait()
                    j = NACC + NACC * t + k
                    pltpu.async_copy(
                        feat_hbm.at[idx_v.at[r0 + j]], accs[k], sems[k], add=True
                    )
                return carry2

            lax.fori_loop(0, nfull, t_body, 0)
            for k in range(ntail):
                pltpu.make_async_copy(
                    feat_hbm.at[idx_v.at[r0]], accs[k], sems[k]
                ).wait()
                j = NACC + NACC * nfull + k
                pltpu.async_copy(
                    feat_hbm.at[idx_v.at[r0 + j]], accs[k], sems[k], add=True
                )
            for k in range(NACC):
                pltpu.make_async_copy(
                    feat_hbm.at[idx_v.at[r0]], accs[k], sems[k]
                ).wait()

            def red_body(c4, carry3):
                for cc in range(CU):
                    for v in range(VSEG):
                        sl = pl.ds(v * L, L)
                        acc = accs[0][c4 * CU + cc, sl]
                        for k in range(1, NACC):
                            acc = acc + accs[k][c4 * CU + cc, sl]
                        accs[0][c4 * CU + cc, sl] = acc * scale
                return carry3

            lax.fori_loop(0, G // CU, red_body, 0)
            row0 = w * BPW + g * G

            @pl.when(row0 + G <= B)
            def _():
                pltpu.sync_copy(accs[0], out_hbm.at[pl.ds(row0, G)])

            @pl.when(row0 + G > B)
            def _():
                pltpu.sync_copy(
                    accs[0].at[pl.ds(0, REM)], out_hbm.at[pl.ds(B - REM, REM)]
                )

            return carry

        lax.fori_loop(0, ng, g_body, 0)

    return body(idx, features)


def kernel(nodes, adj, features):
    nodes = nodes.astype(jnp.int32)
    adj = adj.astype(jnp.int32)
    idx = jnp.concatenate([nodes[:, None], adj], axis=1)
    pad = (
        jnp.arange(BPAD - B, dtype=jnp.int32)[:, None]
        + G * jnp.arange(K, dtype=jnp.int32)[None, :]
    )
    idx = jnp.concatenate([idx, pad], axis=0)
    idx = idx.reshape(NW, NG, G, K).transpose(0, 1, 3, 2).reshape(NW, NR, G)
    return _sc_aggregate(idx, features)
